# Initial kernel scaffold; baseline (speedup 1.0000x reference)
import jax
import jax.numpy as jnp
from jax.experimental import pallas as pl


def kernel(x, edge_index, adj_values, W1, b1, W2, b2, Wc, bc):
    raise NotImplementedError("write your pallas kernel here")


# speedup vs baseline: 4.2713x; 4.2713x over previous
import functools

import jax
import jax.numpy as jnp
from jax import lax
from jax.experimental import pallas as pl
from jax.experimental.pallas import tpu as pltpu
from jax.experimental.pallas import tpu_sc as plsc

_N = 10000
_E = 320000
_D = 128
_LANES = 16

_NC = 2
_NS = 16
_NW = _NC * _NS

_CHUNK = 128
_NCHUNKS = -(-_E // (_NW * _CHUNK))
_EPAD = _NW * _NCHUNKS * _CHUNK

_BM = 400


def _mm_bias_body(x_ref, w_ref, b_ref, o_ref):
    o_ref[...] = (
        jnp.dot(x_ref[...], w_ref[...], preferred_element_type=jnp.float32)
        + b_ref[...]
    )


def _dense_bias(x, W, b):
    m, k = x.shape
    n = W.shape[1]
    return pl.pallas_call(
        _mm_bias_body,
        grid=(m // _BM,),
        in_specs=[
            pl.BlockSpec((_BM, k), lambda i: (i, 0)),
            pl.BlockSpec((k, n), lambda i: (0, 0)),
            pl.BlockSpec((1, n), lambda i: (0, 0)),
        ],
        out_specs=pl.BlockSpec((_BM, n), lambda i: (i, 0)),
        out_shape=jax.ShapeDtypeStruct((m, n), jnp.float32),
    )(x, W, b.reshape(1, n))


def _comb_mm_body(relu_flag, p_ref, w_ref, b_ref, o_ref):
    h = p_ref[0] + p_ref[1]
    if relu_flag:
        h = jnp.maximum(h, 0.0)
    o_ref[...] = (
        jnp.dot(h, w_ref[...], preferred_element_type=jnp.float32) + b_ref[...]
    )


def _combine_dense_bias(p, W, b, relu):
    m = p.shape[1]
    k = p.shape[2]
    n = W.shape[1]
    return pl.pallas_call(
        functools.partial(_comb_mm_body, relu),
        grid=(m // _BM,),
        in_specs=[
            pl.BlockSpec((2, _BM, k), lambda i: (0, i, 0)),
            pl.BlockSpec((k, n), lambda i: (0, 0)),
            pl.BlockSpec((1, n), lambda i: (0, 0)),
        ],
        out_specs=pl.BlockSpec((_BM, n), lambda i: (i, 0)),
        out_shape=jax.ShapeDtypeStruct((m, n), jnp.float32),
    )(p, W, b.reshape(1, n))


_sc_mesh = plsc.VectorSubcoreMesh(core_axis_name="c", subcore_axis_name="s")


@functools.partial(
    pl.kernel,
    mesh=_sc_mesh,
    out_type=jax.ShapeDtypeStruct((_NC, _N, _D), jnp.float32),
    scratch_types=[
        pltpu.VMEM((_NCHUNKS, _CHUNK), jnp.int32),
        pltpu.VMEM((_NCHUNKS, _CHUNK), jnp.int32),
        pltpu.VMEM((_NCHUNKS, _CHUNK), jnp.float32),
        pltpu.VMEM((_CHUNK, _D), jnp.float32),
        pltpu.VMEM_SHARED((_N, _D), jnp.float32),
        pltpu.SemaphoreType.DMA,
    ],
)
def _spmm_sc(s_hbm, row_hbm, col_hbm, adj_hbm, z_hbm, out_hbm,
             colv, rowv, adjv, gbuf, acc, sem):
    c = lax.axis_index("c")
    sid = lax.axis_index("s")
    wid = sid * _NC + c

    @pl.when(sid == 0)
    def _():
        pltpu.sync_copy(z_hbm, acc)

    pltpu.sync_copy(col_hbm.at[wid], colv)
    pltpu.sync_copy(row_hbm.at[wid], rowv)
    pltpu.sync_copy(adj_hbm.at[wid], adjv)
    plsc.subcore_barrier()

    def chunk_body(i, carry):
        pltpu.async_copy(s_hbm.at[colv.at[i]], gbuf, sem).wait()

        def group_body(g, carry2):
            av16 = adjv[i, pl.ds(g * _LANES, _LANES)]
            for l in range(_LANES):
                av = jnp.full((_LANES,), av16[l], jnp.float32)
                e = g * _LANES + l
                for j in range(_D // _LANES):
                    sl = pl.ds(j * _LANES, _LANES)
                    gbuf[e, sl] = gbuf[e, sl] * av
            return carry2

        lax.fori_loop(0, _CHUNK // _LANES, group_body, 0)

        pltpu.sync_copy(gbuf, acc.at[rowv.at[i]], add=True)
        return carry

    lax.fori_loop(0, _NCHUNKS, chunk_body, 0)
    plsc.subcore_barrier()

    @pl.when(sid == 0)
    def _():
        pltpu.sync_copy(acc, out_hbm.at[c])


def kernel(x, edge_index, adj_values, W1, b1, W2, b2, Wc, bc):
    row = edge_index[0].astype(jnp.int32)
    col = edge_index[1].astype(jnp.int32)
    adj = adj_values.astype(jnp.float32)

    pad = _EPAD - _E
    row_p = jnp.concatenate([row, jnp.zeros((pad,), jnp.int32)]).reshape(
        _NW, _NCHUNKS, _CHUNK)
    col_p = jnp.concatenate([col, jnp.zeros((pad,), jnp.int32)]).reshape(
        _NW, _NCHUNKS, _CHUNK)
    adj_p = jnp.concatenate([adj, jnp.zeros((pad,), jnp.float32)]).reshape(
        _NW, _NCHUNKS, _CHUNK)
    z = jnp.zeros((_N, _D), jnp.float32)

    s1 = _dense_bias(x, W1, b1)
    p = _spmm_sc(s1, row_p, col_p, adj_p, z)
    s2 = _combine_dense_bias(p, W2, b2, relu=True)
    q = _spmm_sc(s2, row_p, col_p, adj_p, z)
    logits = _combine_dense_bias(q, Wc, bc, relu=False)
    return logits
